# Initial kernel scaffold; baseline (speedup 1.0000x reference)
import jax
import jax.numpy as jnp
from jax.experimental import pallas as pl


def kernel(token_ids, token_table, pos_table, gamma, beta):
    raise NotImplementedError("write your pallas kernel here")


# speedup vs baseline: 1.4923x; 1.4923x over previous
import functools

import jax
import jax.numpy as jnp
from jax import lax
from jax.experimental import pallas as pl
from jax.experimental.pallas import tpu as pltpu
from jax.experimental.pallas import tpu_sc as plsc

NC = 2
NS = 16
NW = NC * NS
LANES = 16

B = 1024
L = 200
E = 128
NV = E // LANES

CHUNK = 100
ROWS = B * L
NCHUNKS = ROWS // CHUNK
CHUNKS_PER_W = NCHUNKS // NW

_EPS = 1e-5
_RSQRT_MAGIC = 0x5F3759DF


def _rsqrt_vec(v16):
    i = lax.bitcast_convert_type(v16, jnp.int32)
    y = lax.bitcast_convert_type(jnp.int32(_RSQRT_MAGIC) - (i >> 1), jnp.float32)
    half = v16 * jnp.float32(0.5)
    for _ in range(3):
        y = y * (jnp.float32(1.5) - half * y * y)
    return y


@functools.lru_cache(maxsize=1)
def _make_kernel():
    mesh = plsc.VectorSubcoreMesh(core_axis_name="c", subcore_axis_name="s")

    @functools.partial(
        pl.kernel,
        mesh=mesh,
        compiler_params=pltpu.CompilerParams(needs_layout_passes=False),
        out_type=jax.ShapeDtypeStruct((NCHUNKS, CHUNK, E), jnp.float32),
        scratch_types=[
            pltpu.VMEM((CHUNK,), jnp.int32),
            pltpu.VMEM((CHUNK, E), jnp.float32),
            pltpu.VMEM((L, E), jnp.float32),
            pltpu.VMEM((E,), jnp.float32),
            pltpu.VMEM((E,), jnp.float32),
            pltpu.SemaphoreType.DMA,
        ],
    )
    def emb_ln(ids_hbm, tab_hbm, pos_hbm, gam_hbm, bet_hbm, out_hbm,
               idx_v, rows_v, pos_v, gam_v, bet_v, sem):
        wid = lax.axis_index("s") * NC + lax.axis_index("c")

        pltpu.sync_copy(pos_hbm.at[pl.ds(0, L)], pos_v)
        pltpu.sync_copy(gam_hbm, gam_v)
        pltpu.sync_copy(bet_hbm, bet_v)
        gs = [gam_v[pl.ds(v * LANES, LANES)] for v in range(NV)]
        bs = [bet_v[pl.ds(v * LANES, LANES)] for v in range(NV)]

        inv_e = jnp.float32(1.0 / E)

        def chunk_body(c, carry):
            r = wid * CHUNKS_PER_W + c
            pltpu.sync_copy(ids_hbm.at[r], idx_v)
            pltpu.async_copy(tab_hbm.at[idx_v], rows_v, sem).wait()
            pbase = (c % 2) * CHUNK

            def row_body(j, carry2):
                pj = pbase + j
                xs = []
                for v in range(NV):
                    x = (rows_v[j, pl.ds(v * LANES, LANES)]
                         + pos_v[pj, pl.ds(v * LANES, LANES)])
                    xs.append(x)
                s = ((xs[0] + xs[1]) + (xs[2] + xs[3])) + \
                    ((xs[4] + xs[5]) + (xs[6] + xs[7]))
                q = ((xs[0] * xs[0] + xs[1] * xs[1])
                     + (xs[2] * xs[2] + xs[3] * xs[3])) + \
                    ((xs[4] * xs[4] + xs[5] * xs[5])
                     + (xs[6] * xs[6] + xs[7] * xs[7]))
                mean = plsc.cumsum(s)[LANES - 1] * inv_e
                var = plsc.cumsum(q)[LANES - 1] * inv_e - mean * mean
                vv = jnp.full((LANES,), var + jnp.float32(_EPS), jnp.float32)
                inv = _rsqrt_vec(vv)
                mvec = jnp.full((LANES,), mean, jnp.float32)
                for v in range(NV):
                    o = (xs[v] - mvec) * (inv * gs[v]) + bs[v]
                    rows_v[j, pl.ds(v * LANES, LANES)] = o
                return carry2

            lax.fori_loop(0, CHUNK, row_body, 0, unroll=False)
            pltpu.sync_copy(rows_v, out_hbm.at[r])
            return carry

        lax.fori_loop(0, CHUNKS_PER_W, chunk_body, 0, unroll=False)

    return emb_ln


def kernel(token_ids, token_table, pos_table, gamma, beta):
    ids2d = token_ids.astype(jnp.int32).reshape(NCHUNKS, CHUNK)
    out = _make_kernel()(ids2d, token_table, pos_table, gamma, beta)
    return out.reshape(B, L, E)


# speedup vs baseline: 1.5624x; 1.0470x over previous
import functools

import jax
import jax.numpy as jnp
from jax import lax
from jax.experimental import pallas as pl
from jax.experimental.pallas import tpu as pltpu
from jax.experimental.pallas import tpu_sc as plsc

NC = 2
NS = 16
NW = NC * NS
LANES = 16

B = 1024
L = 200
E = 128
NV = E // LANES

CHUNK = 100
ROWS = B * L
NCHUNKS = ROWS // CHUNK
CHUNKS_PER_W = NCHUNKS // NW

_EPS = 1e-5
_RSQRT_MAGIC = 0x5F3759DF


def _rsqrt_vec(v16):
    i = lax.bitcast_convert_type(v16, jnp.int32)
    y = lax.bitcast_convert_type(jnp.int32(_RSQRT_MAGIC) - (i >> 1), jnp.float32)
    half = v16 * jnp.float32(0.5)
    for _ in range(3):
        y = y * (jnp.float32(1.5) - half * y * y)
    return y


@functools.lru_cache(maxsize=1)
def _make_kernel():
    mesh = plsc.VectorSubcoreMesh(core_axis_name="c", subcore_axis_name="s")

    @functools.partial(
        pl.kernel,
        mesh=mesh,
        compiler_params=pltpu.CompilerParams(needs_layout_passes=False),
        out_type=jax.ShapeDtypeStruct((NCHUNKS, CHUNK, E), jnp.float32),
        scratch_types=[
            pltpu.VMEM((CHUNKS_PER_W, CHUNK), jnp.int32),
            pltpu.VMEM((CHUNK, E), jnp.float32),
            pltpu.VMEM((L, E), jnp.float32),
            pltpu.VMEM((E,), jnp.float32),
            pltpu.VMEM((E,), jnp.float32),
            pltpu.SemaphoreType.DMA,
        ],
    )
    def emb_ln(ids_hbm, tab_hbm, pos_hbm, gam_hbm, bet_hbm, out_hbm,
               idx_v, rows_v, pos_v, gam_v, bet_v, sem):
        wid = lax.axis_index("s") * NC + lax.axis_index("c")

        pltpu.sync_copy(ids_hbm.at[wid], idx_v)
        pltpu.sync_copy(pos_hbm.at[pl.ds(0, L)], pos_v)
        pltpu.sync_copy(gam_hbm, gam_v)
        pltpu.sync_copy(bet_hbm, bet_v)
        gs = [gam_v[pl.ds(v * LANES, LANES)] for v in range(NV)]
        bs = [bet_v[pl.ds(v * LANES, LANES)] for v in range(NV)]

        inv_e = jnp.float32(1.0 / E)

        def chunk_body(c, carry):
            r = wid * CHUNKS_PER_W + c
            pltpu.async_copy(tab_hbm.at[idx_v.at[c]], rows_v, sem).wait()
            pbase = (c % 2) * CHUNK

            def row_body(j, carry2):
                pj = pbase + j
                xs = []
                for v in range(NV):
                    x = (rows_v[j, pl.ds(v * LANES, LANES)]
                         + pos_v[pj, pl.ds(v * LANES, LANES)])
                    xs.append(x)
                s = ((xs[0] + xs[1]) + (xs[2] + xs[3])) + \
                    ((xs[4] + xs[5]) + (xs[6] + xs[7]))
                q = ((xs[0] * xs[0] + xs[1] * xs[1])
                     + (xs[2] * xs[2] + xs[3] * xs[3])) + \
                    ((xs[4] * xs[4] + xs[5] * xs[5])
                     + (xs[6] * xs[6] + xs[7] * xs[7]))
                mean = plsc.cumsum(s)[LANES - 1] * inv_e
                var = plsc.cumsum(q)[LANES - 1] * inv_e - mean * mean
                vv = jnp.full((LANES,), var + jnp.float32(_EPS), jnp.float32)
                inv = _rsqrt_vec(vv)
                mvec = jnp.full((LANES,), mean, jnp.float32)
                for v in range(NV):
                    o = (xs[v] - mvec) * (inv * gs[v]) + bs[v]
                    rows_v[j, pl.ds(v * LANES, LANES)] = o
                return carry2

            lax.fori_loop(0, CHUNK, row_body, 0, unroll=4)
            pltpu.sync_copy(rows_v, out_hbm.at[r])
            return carry

        lax.fori_loop(0, CHUNKS_PER_W, chunk_body, 0, unroll=False)

    return emb_ln


def kernel(token_ids, token_table, pos_table, gamma, beta):
    ids3d = token_ids.astype(jnp.int32).reshape(NW, CHUNKS_PER_W, CHUNK)
    out = _make_kernel()(ids3d, token_table, pos_table, gamma, beta)
    return out.reshape(B, L, E)


# speedup vs baseline: 1.8471x; 1.1822x over previous
import functools

import jax
import jax.numpy as jnp
from jax import lax
from jax.experimental import pallas as pl
from jax.experimental.pallas import tpu as pltpu
from jax.experimental.pallas import tpu_sc as plsc

NC = 2
NS = 16
NW = NC * NS
LANES = 16

B = 1024
L = 200
E = 128
NV = E // LANES

CHUNK = 100
ROWS = B * L
NCHUNKS = ROWS // CHUNK
CHUNKS_PER_W = NCHUNKS // NW
NPAIRS = CHUNKS_PER_W // 2

_EPS = 1e-5
_RSQRT_MAGIC = 0x5F3759DF


def _rsqrt_vec(v16):
    i = lax.bitcast_convert_type(v16, jnp.int32)
    y = lax.bitcast_convert_type(jnp.int32(_RSQRT_MAGIC) - (i >> 1), jnp.float32)
    half = v16 * jnp.float32(0.5)
    for _ in range(3):
        y = y * (jnp.float32(1.5) - half * y * y)
    return y


@functools.lru_cache(maxsize=1)
def _make_kernel():
    mesh = plsc.VectorSubcoreMesh(core_axis_name="c", subcore_axis_name="s")

    @functools.partial(
        pl.kernel,
        mesh=mesh,
        compiler_params=pltpu.CompilerParams(needs_layout_passes=False),
        out_type=jax.ShapeDtypeStruct((NCHUNKS, CHUNK, E), jnp.float32),
        scratch_types=[
            pltpu.VMEM((CHUNKS_PER_W, CHUNK), jnp.int32),
            pltpu.VMEM((CHUNK, E), jnp.float32),
            pltpu.VMEM((CHUNK, E), jnp.float32),
            pltpu.VMEM((CHUNK, E), jnp.float32),
            pltpu.VMEM((CHUNK, E), jnp.float32),
            pltpu.VMEM((L, E), jnp.float32),
            pltpu.VMEM((E,), jnp.float32),
            pltpu.VMEM((E,), jnp.float32),
            pltpu.SemaphoreType.DMA,
            pltpu.SemaphoreType.DMA,
            pltpu.SemaphoreType.DMA,
            pltpu.SemaphoreType.DMA,
        ],
    )
    def emb_ln(ids_hbm, tab_hbm, pos_hbm, gam_hbm, bet_hbm, out_hbm,
               idx_v, g0, g1, o0, o1, pos_v, gam_v, bet_v,
               gs0, gs1, os0, os1):
        wid = lax.axis_index("s") * NC + lax.axis_index("c")
        gb, ob = [g0, g1], [o0, o1]
        gsem, osem = [gs0, gs1], [os0, os1]

        pltpu.sync_copy(ids_hbm.at[wid], idx_v)
        pltpu.sync_copy(pos_hbm.at[pl.ds(0, L)], pos_v)
        pltpu.sync_copy(gam_hbm, gam_v)
        pltpu.sync_copy(bet_hbm, bet_v)
        gs = [gam_v[pl.ds(v * LANES, LANES)] for v in range(NV)]
        bs = [bet_v[pl.ds(v * LANES, LANES)] for v in range(NV)]

        inv_e = jnp.float32(1.0 / E)

        def issue_gather(c, b):
            pltpu.async_copy(tab_hbm.at[idx_v.at[c]], gb[b], gsem[b])

        def compute_chunk(c, b):
            pbase = (c % 2) * CHUNK
            src, dst = gb[b], ob[b]

            def row_body(j, carry2):
                pj = pbase + j
                xs = []
                for v in range(NV):
                    x = (src[j, pl.ds(v * LANES, LANES)]
                         + pos_v[pj, pl.ds(v * LANES, LANES)])
                    xs.append(x)
                s = ((xs[0] + xs[1]) + (xs[2] + xs[3])) + \
                    ((xs[4] + xs[5]) + (xs[6] + xs[7]))
                q = ((xs[0] * xs[0] + xs[1] * xs[1])
                     + (xs[2] * xs[2] + xs[3] * xs[3])) + \
                    ((xs[4] * xs[4] + xs[5] * xs[5])
                     + (xs[6] * xs[6] + xs[7] * xs[7]))
                mean = plsc.cumsum(s)[LANES - 1] * inv_e
                var = plsc.cumsum(q)[LANES - 1] * inv_e - mean * mean
                vv = jnp.full((LANES,), var + jnp.float32(_EPS), jnp.float32)
                inv = _rsqrt_vec(vv)
                mvec = jnp.full((LANES,), mean, jnp.float32)
                for v in range(NV):
                    o = (xs[v] - mvec) * (inv * gs[v]) + bs[v]
                    dst[j, pl.ds(v * LANES, LANES)] = o
                return carry2

            lax.fori_loop(0, CHUNK, row_body, 0, unroll=4)

        issue_gather(0, 0)

        def pair_body(p, carry):
            for b in range(2):
                c = 2 * p + b

                @pl.when(c < CHUNKS_PER_W - 1)
                def _():
                    issue_gather(c + 1, 1 - b)

                pltpu.make_async_copy(tab_hbm.at[idx_v.at[c]], gb[b],
                                      gsem[b]).wait()

                @pl.when(c >= 2)
                def _():
                    pltpu.make_async_copy(ob[b], out_hbm.at[0], osem[b]).wait()

                compute_chunk(c, b)
                pltpu.async_copy(ob[b], out_hbm.at[wid * CHUNKS_PER_W + c],
                                 osem[b])
            return carry

        lax.fori_loop(0, NPAIRS, pair_body, 0, unroll=False)

        pltpu.make_async_copy(ob[0], out_hbm.at[0], osem[0]).wait()
        pltpu.make_async_copy(ob[1], out_hbm.at[0], osem[1]).wait()

    return emb_ln


def kernel(token_ids, token_table, pos_table, gamma, beta):
    ids3d = token_ids.astype(jnp.int32).reshape(NW, CHUNKS_PER_W, CHUNK)
    out = _make_kernel()(ids3d, token_table, pos_table, gamma, beta)
    return out.reshape(B, L, E)
